# Initial kernel scaffold; baseline (speedup 1.0000x reference)
import jax
import jax.numpy as jnp
from jax.experimental import pallas as pl


def kernel(x, word_table, temp_table):
    raise NotImplementedError("write your pallas kernel here")


# speedup vs baseline: 1.0207x; 1.0207x over previous
import jax
import jax.numpy as jnp
from jax.experimental import pallas as pl


def _add_kernel(x_ref, word_ref, temp_ref, out_ref):
    out_ref[...] = (
        x_ref[...]
        + word_ref[...][None, :, None, :]
        + temp_ref[...][None, None, :, :]
    )


def kernel(x, word_table, temp_table):
    batch, num_words, time_len, hidden = x.shape
    temp_slice = temp_table[:time_len]
    return pl.pallas_call(
        _add_kernel,
        grid=(batch,),
        in_specs=[
            pl.BlockSpec((1, num_words, time_len, hidden), lambda b: (b, 0, 0, 0)),
            pl.BlockSpec((num_words, hidden), lambda b: (0, 0)),
            pl.BlockSpec((time_len, hidden), lambda b: (0, 0)),
        ],
        out_specs=pl.BlockSpec(
            (1, num_words, time_len, hidden), lambda b: (b, 0, 0, 0)
        ),
        out_shape=jax.ShapeDtypeStruct(x.shape, x.dtype),
    )(x, word_table, temp_slice)
